# baseline (speedup 1.0000x reference)
import functools

import jax
import jax.numpy as jnp
from jax import lax
from jax.experimental import pallas as pl
from jax.experimental.pallas import tpu as pltpu
from jax.experimental.pallas import tpu_sc as plsc

HIDDEN = 768
NV = HIDDEN // 16

NC = 2
NS = 16
NW = NC * NS

CHUNK = 64


def _rsqrt_vec(v):
    i = plsc.bitcast(v, jnp.int32)
    i = jnp.int32(0x5F3759DF) - (i >> 1)
    y = plsc.bitcast(i, jnp.float32)
    for _ in range(3):
        y = y * (1.5 - 0.5 * v * y * y)
    return y


def _make_sc_kernel(tokens, seq):
    tpw = tokens // NW
    nchunk = tpw // CHUNK
    mesh = plsc.VectorSubcoreMesh(
        core_axis_name="c", subcore_axis_name="s", num_cores=NC, num_subcores=NS
    )

    @functools.partial(
        pl.kernel,
        mesh=mesh,
        out_type=jax.ShapeDtypeStruct((tokens, HIDDEN), jnp.float32),
        scratch_types=[
            pltpu.VMEM((CHUNK,), jnp.int32),
            pltpu.VMEM((CHUNK, HIDDEN), jnp.float32),
            pltpu.VMEM((CHUNK, HIDDEN), jnp.float32),
            pltpu.VMEM((HIDDEN,), jnp.float32),
            pltpu.VMEM((HIDDEN,), jnp.float32),
            pltpu.VMEM((HIDDEN,), jnp.float32),
            pltpu.SemaphoreType.DMA,
        ],
        compiler_params=pltpu.CompilerParams(needs_layout_passes=False),
    )
    def emb_kernel(ids_hbm, word_hbm, pos_hbm, tt_hbm, g_hbm, b_hbm, out_hbm,
                   idx_v, wrows, prows, ttv, gv, bv, sem):
        wid = lax.axis_index("s") * NC + lax.axis_index("c")
        base = wid * tpw
        pos_base = lax.rem(base, seq)
        pltpu.sync_copy(tt_hbm, ttv)
        pltpu.sync_copy(g_hbm, gv)
        pltpu.sync_copy(b_hbm, bv)

        def chunk_body(k, carry):
            tok = base + k * CHUNK
            p0 = pos_base + k * CHUNK
            pltpu.sync_copy(ids_hbm.at[pl.ds(tok, CHUNK)], idx_v)
            pltpu.async_copy(word_hbm.at[idx_v], wrows, sem).wait()
            pltpu.sync_copy(pos_hbm.at[pl.ds(p0, CHUNK)], prows)

            def token_body(t, carry2):
                acc = jnp.zeros((16,), jnp.float32)
                acc2 = jnp.zeros((16,), jnp.float32)
                for j in range(NV):
                    sl = pl.ds(j * 16, 16)
                    v = wrows[t, sl] + prows[t, sl] + ttv[sl]
                    wrows[t, sl] = v
                    acc = acc + v
                    acc2 = acc2 + v * v
                mean = jnp.sum(acc) * (1.0 / HIDDEN)
                var = jnp.sum(acc2) * (1.0 / HIDDEN) - mean * mean
                mv = jnp.full((16,), mean, jnp.float32)
                rv = _rsqrt_vec(jnp.full((16,), var + 1e-12, jnp.float32))
                for j in range(NV):
                    sl = pl.ds(j * 16, 16)
                    x = wrows[t, sl]
                    wrows[t, sl] = (x - mv) * rv * gv[sl] + bv[sl]
                return carry2

            lax.fori_loop(0, CHUNK, token_body, 0)
            pltpu.sync_copy(wrows, out_hbm.at[pl.ds(tok, CHUNK)])
            return carry

        lax.fori_loop(0, nchunk, chunk_body, 0)

    return emb_kernel


def kernel(input_ids, word_table, token_type_table, pos_table, ln_gamma, ln_beta):
    batch, seq = input_ids.shape
    tokens = batch * seq
    ids = input_ids.reshape(tokens)
    sc = _make_sc_kernel(tokens, seq)
    out = sc(ids, word_table, pos_table, token_type_table[0], ln_gamma, ln_beta)
    return out.reshape(batch, seq, HIDDEN)


# speedup vs baseline: 1.3798x; 1.3798x over previous
import functools

import jax
import jax.numpy as jnp
from jax import lax
from jax.experimental import pallas as pl
from jax.experimental.pallas import tpu as pltpu
from jax.experimental.pallas import tpu_sc as plsc

HIDDEN = 768
NV = HIDDEN // 16

NC = 2
NS = 16
NW = NC * NS

TC = 32
NSTEP = 32
NBUF = 3


def _rsqrt_vec(v):
    i = plsc.bitcast(v, jnp.int32)
    i = jnp.int32(0x5F3759DF) - (i >> 1)
    y = plsc.bitcast(i, jnp.float32)
    for _ in range(3):
        y = y * (1.5 - 0.5 * v * y * y)
    return y


def _make_sc_kernel(batch, seq):
    tokens = batch * seq
    ppw = seq // NW
    tpw = batch * ppw
    assert tpw == TC * NSTEP
    mesh = plsc.VectorSubcoreMesh(
        core_axis_name="c", subcore_axis_name="s", num_cores=NC, num_subcores=NS
    )

    @functools.partial(
        pl.kernel,
        mesh=mesh,
        out_type=jax.ShapeDtypeStruct((tokens, HIDDEN), jnp.float32),
        scratch_types=[
            pltpu.VMEM((tpw,), jnp.int32),
            pltpu.VMEM((TC, HIDDEN), jnp.float32),
            pltpu.VMEM((TC, HIDDEN), jnp.float32),
            pltpu.VMEM((TC, HIDDEN), jnp.float32),
            pltpu.VMEM((TC, HIDDEN), jnp.float32),
            pltpu.VMEM((HIDDEN,), jnp.float32),
            pltpu.VMEM((HIDDEN,), jnp.float32),
            pltpu.VMEM((HIDDEN,), jnp.float32),
            pltpu.SemaphoreType.DMA,
            pltpu.SemaphoreType.DMA,
            pltpu.SemaphoreType.DMA,
            pltpu.SemaphoreType.DMA,
            pltpu.SemaphoreType.DMA,
            pltpu.SemaphoreType.DMA,
        ],
        compiler_params=pltpu.CompilerParams(needs_layout_passes=False),
    )
    def emb_kernel(ids_hbm, word_hbm, pos_hbm, tt_hbm, g_hbm, b_hbm, out_hbm,
                   ids_v, w0, w1, w2, prows, ttv, gv, bv,
                   g0, g1, g2, o0, o1, o2):
        bufs = (w0, w1, w2)
        gsems = (g0, g1, g2)
        osems = (o0, o1, o2)
        wid = lax.axis_index("s") * NC + lax.axis_index("c")
        pbase = wid * ppw

        pltpu.sync_copy(tt_hbm, ttv)
        pltpu.sync_copy(g_hbm, gv)
        pltpu.sync_copy(b_hbm, bv)
        for b in range(batch):
            pltpu.sync_copy(
                ids_hbm.at[pl.ds(b * seq + pbase, ppw)],
                ids_v.at[pl.ds(b * ppw, ppw)],
            )

        def ids_off(s):
            return (s & 3) * ppw + (s >> 2) * TC

        def out_off(s):
            return (s & 3) * seq + pbase + (s >> 2) * TC

        def gather(s, k):
            pltpu.async_copy(
                word_hbm.at[ids_v.at[pl.ds(ids_off(s), TC)]], bufs[k], gsems[k]
            )

        def gather_wait(s, k):
            pltpu.make_async_copy(
                word_hbm.at[ids_v.at[pl.ds(ids_off(s), TC)]], bufs[k], gsems[k]
            ).wait()

        def out_copy(s, k):
            pltpu.async_copy(
                bufs[k], out_hbm.at[pl.ds(out_off(s), TC)], osems[k]
            )

        def out_wait(s, k):
            pltpu.make_async_copy(
                bufs[k], out_hbm.at[pl.ds(out_off(s), TC)], osems[k]
            ).wait()

        def compute(buf):
            def token_body(t, carry):
                acc = jnp.zeros((16,), jnp.float32)
                acc2 = jnp.zeros((16,), jnp.float32)
                for j in range(NV):
                    sl = pl.ds(j * 16, 16)
                    v = buf[t, sl] + prows[t, sl]
                    buf[t, sl] = v
                    acc = acc + v
                    acc2 = acc2 + v * v
                mean = jnp.sum(acc) * (1.0 / HIDDEN)
                var = jnp.sum(acc2) * (1.0 / HIDDEN) - mean * mean
                mv = jnp.full((16,), mean, jnp.float32)
                rv = _rsqrt_vec(jnp.full((16,), var + 1e-12, jnp.float32))
                for j in range(NV):
                    sl = pl.ds(j * 16, 16)
                    x = buf[t, sl]
                    buf[t, sl] = (x - mv) * rv * gv[sl] + bv[sl]
                return carry

            lax.fori_loop(0, TC, token_body, 0)

        def load_pos(pc):
            pltpu.sync_copy(pos_hbm.at[pl.ds(pbase + pc * TC, TC)], prows)

            def fold(t, carry):
                for j in range(NV):
                    sl = pl.ds(j * 16, 16)
                    prows[t, sl] = prows[t, sl] + ttv[sl]
                return carry

            lax.fori_loop(0, TC, fold, 0)

        def step(s, spy):
            k = spy
            kn = (spy + 1) % NBUF
            gather_wait(s, k)
            if isinstance(s, int):
                if s >= 2:
                    out_wait(s - 2, kn)
                if s + 1 < NSTEP:
                    gather(s + 1, kn)
            else:
                @pl.when(s >= 2)
                def _():
                    out_wait(s - 2, kn)

                @pl.when(s + 1 < NSTEP)
                def _():
                    gather(s + 1, kn)
            @pl.when((s & 3) == 0)
            def _():
                load_pos(s >> 2)

            compute(bufs[k])
            out_copy(s, k)

        gather(0, 0)
        def loop_body(i, carry):
            s = i * NBUF
            step(s, 0)
            step(s + 1, 1)
            step(s + 2, 2)
            return carry

        lax.fori_loop(0, NSTEP // NBUF, loop_body, 0)
        step(30, 0)
        step(31, 1)
        out_wait(30, 0)
        out_wait(31, 1)

    return emb_kernel


def kernel(input_ids, word_table, token_type_table, pos_table, ln_gamma, ln_beta):
    batch, seq = input_ids.shape
    tokens = batch * seq
    ids = input_ids.reshape(tokens)
    sc = _make_sc_kernel(batch, seq)
    out = sc(ids, word_table, pos_table, token_type_table[0], ln_gamma, ln_beta)
    return out.reshape(batch, seq, HIDDEN)
